# Initial kernel scaffold; baseline (speedup 1.0000x reference)
import jax
import jax.numpy as jnp
from jax.experimental import pallas as pl


def kernel(input_ids, positions, out_cache_loc, embed_tokens):
    raise NotImplementedError("write your pallas kernel here")


# speedup vs baseline: 1.6274x; 1.6274x over previous
import functools

import jax
import jax.numpy as jnp
from jax import lax
from jax.experimental import pallas as pl
from jax.experimental.pallas import tpu as pltpu
from jax.experimental.pallas import tpu_sc as plsc

D_MODEL = 2048
NTOK = 4 * 4096
NC, NS = 2, 16
NW = NC * NS
TOK_PER_W = NTOK // NW
K = 32
NCHUNK = TOK_PER_W // K

_mesh = plsc.VectorSubcoreMesh(
    core_axis_name="c", subcore_axis_name="s", num_cores=NC, num_subcores=NS
)


@functools.partial(
    pl.kernel,
    out_type=jax.ShapeDtypeStruct((NTOK, D_MODEL), jnp.float32),
    mesh=_mesh,
    scratch_types=[
        pltpu.VMEM((TOK_PER_W,), jnp.int32),
        pltpu.VMEM((K, D_MODEL), jnp.float32),
        pltpu.SemaphoreType.DMA,
    ],
)
def _gather(ids_hbm, table_hbm, out_hbm, idx_v, rows_v, sem):
    wid = lax.axis_index("s") * NC + lax.axis_index("c")
    base = pl.multiple_of(wid * TOK_PER_W, TOK_PER_W)
    pltpu.sync_copy(ids_hbm.at[pl.ds(base, TOK_PER_W)], idx_v)

    def body(j, carry):
        off = pl.multiple_of(j * K, K)
        pltpu.async_copy(
            table_hbm.at[idx_v.at[pl.ds(off, K)]], rows_v, sem
        ).wait()
        pltpu.sync_copy(rows_v, out_hbm.at[pl.ds(base + off, K)])
        return carry

    lax.fori_loop(0, NCHUNK, body, 0)


def kernel(input_ids, positions, out_cache_loc, embed_tokens):
    ids = input_ids.reshape(-1)
    out = _gather(ids, embed_tokens)
    return out.reshape(*input_ids.shape, D_MODEL)


# speedup vs baseline: 1.7229x; 1.0587x over previous
import functools

import jax
import jax.numpy as jnp
from jax import lax
from jax.experimental import pallas as pl
from jax.experimental.pallas import tpu as pltpu
from jax.experimental.pallas import tpu_sc as plsc

D_MODEL = 2048
NTOK = 4 * 4096
NC, NS = 2, 16
NW = NC * NS
TOK_PER_W = NTOK // NW
K = 8
NBUF = 4
NCHUNK = TOK_PER_W // K
NITER = NCHUNK // NBUF

_mesh = plsc.VectorSubcoreMesh(
    core_axis_name="c", subcore_axis_name="s", num_cores=NC, num_subcores=NS
)


@functools.partial(
    pl.kernel,
    out_type=jax.ShapeDtypeStruct((NTOK, D_MODEL), jnp.float32),
    mesh=_mesh,
    scratch_types=[
        pltpu.VMEM((TOK_PER_W,), jnp.int32),
        [pltpu.VMEM((K, D_MODEL), jnp.float32) for _ in range(NBUF)],
        [pltpu.SemaphoreType.DMA for _ in range(NBUF)],
        [pltpu.SemaphoreType.DMA for _ in range(NBUF)],
    ],
)
def _gather(ids_hbm, table_hbm, out_hbm, idx_v, rows, gsem, ssem):
    wid = lax.axis_index("s") * NC + lax.axis_index("c")
    base = pl.multiple_of(wid * TOK_PER_W, TOK_PER_W)
    pltpu.sync_copy(ids_hbm.at[pl.ds(base, TOK_PER_W)], idx_v)

    def idx_slice(j):
        return idx_v.at[pl.ds(pl.multiple_of(j * K, 8), K)]

    def out_slice(j):
        return out_hbm.at[pl.ds(base + pl.multiple_of(j * K, 8), K)]

    def g_start(j, b):
        pltpu.async_copy(table_hbm.at[idx_slice(j)], rows[b], gsem[b])

    def g_wait(j, b):
        pltpu.make_async_copy(table_hbm.at[idx_slice(j)], rows[b], gsem[b]).wait()

    def s_start(j, b):
        pltpu.async_copy(rows[b], out_slice(j), ssem[b])

    def s_wait(j, b):
        pltpu.make_async_copy(rows[b], out_slice(j), ssem[b]).wait()

    for b in range(NBUF):
        g_start(b, b)

    def body(i, carry):
        j0 = i * NBUF
        for b in range(NBUF):
            g_wait(j0 + b, b)
            s_start(j0 + b, b)
        for b in range(NBUF):
            s_wait(j0 + b, b)
            g_start(j0 + NBUF + b, b)
        return carry

    lax.fori_loop(0, NITER - 1, body, 0)

    jl = (NITER - 1) * NBUF
    for b in range(NBUF):
        g_wait(jl + b, b)
        s_start(jl + b, b)
    for b in range(NBUF):
        s_wait(jl + b, b)


def kernel(input_ids, positions, out_cache_loc, embed_tokens):
    ids = input_ids.reshape(-1)
    out = _gather(ids, embed_tokens)
    return out.reshape(*input_ids.shape, D_MODEL)
